# Initial kernel scaffold; baseline (speedup 1.0000x reference)
import jax
import jax.numpy as jnp
from jax import lax
from jax.experimental import pallas as pl
from jax.experimental.pallas import tpu as pltpu
from jax.experimental.pallas import tpu_sc as plsc

B = 128
N = 10000
R1 = 80000
R2 = 80000
NSURF = 2000
NMANT = 2000
NPH = 1000
NSMT = 1000
AG = 100.0

NC = 2
NS = 16
NW = NC * NS
BT = B // NW
C = 2000
NCH1 = R1 // C
NCH2 = R2 // C
SPAN = R1 // NS
L = 16

f32 = jnp.float32
i32 = jnp.int32


def _med_body(t_ref, o_ref):
    t = t_ref[...]
    arg = t * 1e-5
    T = 10.0 + 5.0 * jnp.sin(arg)
    den = 1.0e4 * (1.0 + 0.1 * jnp.cos(arg))
    logT = jnp.log(T / 300.0)
    nit = -1.0 / T
    z = jnp.zeros_like(T)
    o_ref[...] = jnp.concatenate([logT, nit, den, z, z, z, z, z], axis=0)


def _sc_body(y_h, scal_h, alpha_h, beta_h, gamma_h, reac1_h, prod1_h,
             r2a_h, r2b_h, prod2_h, surf_h, mant_h, ph_h, smt_h,
             out_h,
             y_t, out_t, cnt_t, idx_a, idx_b, idx_c, par_a, par_b, par_c,
             flg_a, flg_b, scal_t, ph_sp, smt_sp, sem0, sem1, semm):
    cid = lax.axis_index("c")
    sid = lax.axis_index("s")
    tid = cid * NS + sid
    iot = lax.iota(i32, L)
    ones = jnp.ones((L,), f32)
    zf = jnp.zeros((L,), f32)
    sems = (sem0, sem1)

    cps = [
        pltpu.async_copy(y_h.at[pl.ds(tid * BT, BT)], y_t, semm),
        pltpu.async_copy(scal_h.at[pl.ds(tid * BT, BT)], scal_t, semm),
        pltpu.async_copy(ph_h, idx_a.at[0, pl.ds(0, NPH)], semm),
        pltpu.async_copy(smt_h, idx_a.at[1, pl.ds(0, NSMT)], semm),
        pltpu.async_copy(surf_h, idx_b.at[0, pl.ds(0, NSURF)], semm),
        pltpu.async_copy(mant_h, idx_b.at[1, pl.ds(0, NMANT)], semm),
    ]
    for c_ in cps:
        c_.wait()

    def _zero_row(row, nsteps):
        def bd(i, carry):
            out_t[row, pl.ds(pl.multiple_of(i * L, L), L)] = zf
            return carry
        lax.fori_loop(0, nsteps, bd, 0)

    base = sid * SPAN

    def _build_flags(row, nidx, sp_ref):
        _zero_row(0, SPAN // L + 1)

        def bd(i, carry):
            idx = idx_a[row, pl.ds(pl.multiple_of(i * L, L), L)]
            valid = (i * L + iot) < nidx
            loc = idx - base
            m = valid & (loc >= 0) & (loc < SPAN)
            locc = jnp.clip(loc, 0, SPAN)
            plsc.store_scatter(out_t.at[0], [locc], ones, mask=m)
            return carry
        lax.fori_loop(0, (nidx + L - 1) // L, bd, 0)
        pltpu.sync_copy(out_t.at[0, pl.ds(0, SPAN)],
                        sp_ref.at[pl.ds(base, SPAN)])

    _build_flags(0, NPH, ph_sp)
    _build_flags(1, NSMT, smt_sp)
    plsc.subcore_barrier()

    for b in range(BT):
        _zero_row(b, N // L)

    def _zc(i, carry):
        cnt_t[pl.ds(pl.multiple_of(i * L, L), L)] = zf
        return carry
    lax.fori_loop(0, N // L, _zc, 0)

    bspl = [jnp.full((L,), b, i32) for b in range(BT)]

    def _ssum(i, acc):
        idx = idx_b[0, pl.ds(pl.multiple_of(i * L, L), L)]
        plsc.addupdate_scatter(cnt_t, [idx], ones)
        return tuple(acc[b] + plsc.load_gather(y_t, [bspl[b], idx])
                     for b in range(BT))
    acc_s = lax.fori_loop(0, NSURF // L, _ssum, (zf,) * BT)

    def _msum(i, acc):
        idx = idx_b[1, pl.ds(pl.multiple_of(i * L, L), L)]
        return tuple(acc[b] + plsc.load_gather(y_t, [bspl[b], idx])
                     for b in range(BT))
    acc_m = lax.fori_loop(0, NMANT // L, _msum, (zf,) * BT)

    ys = [jnp.full((L,), jnp.sum(acc_s[b]), f32) for b in range(BT)]
    ym = [jnp.full((L,), jnp.sum(acc_m[b]), f32) for b in range(BT)]
    dec = [jnp.minimum(1.0 / (AG * (ys[b] + ym[b])), 1.0) for b in range(BT)]
    Lb = [jnp.full((L,), scal_t[b, 0], f32) for b in range(BT)]
    nit = [jnp.full((L,), scal_t[b, 1], f32) for b in range(BT)]
    den = [jnp.full((L,), scal_t[b, 2], f32) for b in range(BT)]

    def _issue1(g, s, with_smt):
        off = pl.ds(pl.multiple_of(g * C, 8), C)
        sem = sems[s]
        pltpu.async_copy(reac1_h.at[off], idx_a.at[s], sem)
        pltpu.async_copy(prod1_h.at[off], idx_b.at[s], sem)
        pltpu.async_copy(alpha_h.at[off], par_a.at[s], sem)
        pltpu.async_copy(beta_h.at[off], par_b.at[s], sem)
        pltpu.async_copy(gamma_h.at[off], par_c.at[s], sem)
        pltpu.async_copy(ph_sp.at[off], flg_a.at[s], sem)
        if with_smt:
            pltpu.async_copy(smt_sp.at[off], flg_b.at[s], sem)

    def _drain1(g, s, with_smt):
        off = pl.ds(pl.multiple_of(g * C, 8), C)
        sem = sems[s]
        pltpu.make_async_copy(reac1_h.at[off], idx_a.at[s], sem).wait()
        pltpu.make_async_copy(prod1_h.at[off], idx_b.at[s], sem).wait()
        pltpu.make_async_copy(alpha_h.at[off], par_a.at[s], sem).wait()
        pltpu.make_async_copy(beta_h.at[off], par_b.at[s], sem).wait()
        pltpu.make_async_copy(gamma_h.at[off], par_c.at[s], sem).wait()
        pltpu.make_async_copy(ph_sp.at[off], flg_a.at[s], sem).wait()
        if with_smt:
            pltpu.make_async_copy(smt_sp.at[off], flg_b.at[s], sem).wait()

    def _issue2(g, s):
        off = pl.ds(pl.multiple_of(g * C, 8), C)
        off2 = pl.ds(pl.multiple_of(R1 + g * C, 8), C)
        sem = sems[s]
        pltpu.async_copy(r2a_h.at[off], idx_a.at[s], sem)
        pltpu.async_copy(r2b_h.at[off], idx_b.at[s], sem)
        pltpu.async_copy(prod2_h.at[off], idx_c.at[s], sem)
        pltpu.async_copy(alpha_h.at[off2], par_a.at[s], sem)
        pltpu.async_copy(beta_h.at[off2], par_b.at[s], sem)
        pltpu.async_copy(gamma_h.at[off2], par_c.at[s], sem)

    def _drain2(g, s):
        off = pl.ds(pl.multiple_of(g * C, 8), C)
        off2 = pl.ds(pl.multiple_of(R1 + g * C, 8), C)
        sem = sems[s]
        pltpu.make_async_copy(r2a_h.at[off], idx_a.at[s], sem).wait()
        pltpu.make_async_copy(r2b_h.at[off], idx_b.at[s], sem).wait()
        pltpu.make_async_copy(prod2_h.at[off], idx_c.at[s], sem).wait()
        pltpu.make_async_copy(alpha_h.at[off2], par_a.at[s], sem).wait()
        pltpu.make_async_copy(beta_h.at[off2], par_b.at[s], sem).wait()
        pltpu.make_async_copy(gamma_h.at[off2], par_c.at[s], sem).wait()

    def _chunkA1(s, acc):
        def bd(i, acc):
            accg, accl = acc
            off = pl.ds(pl.multiple_of(i * L, L), L)
            reac = idx_a[s, off]
            prod = idx_b[s, off]
            al = par_a[s, off]
            bt = par_b[s, off]
            gm = par_c[s, off]
            phm = flg_a[s, off] > 0.0
            wg = plsc.load_gather(cnt_t, [prod])
            wl = plsc.load_gather(cnt_t, [reac])
            ng, nl = [], []
            for b in range(BT):
                k = al * jnp.exp(bt * Lb[b] + gm * nit[b])
                k = k * jnp.where(phm, dec[b], ones)
                yv = plsc.load_gather(y_t, [bspl[b], reac])
                r = k * yv
                ng.append(accg[b] + wg * r)
                nl.append(accl[b] + wl * r)
            return (tuple(ng), tuple(nl))
        return lax.fori_loop(0, C // L, bd, acc)

    acc = ((zf,) * BT, (zf,) * BT)
    _issue1(0, 0, False)
    _issue1(1, 1, False)

    def _outerA1(h, acc):
        for s in range(2):
            g = h * 2 + s
            _drain1(g, s, False)
            acc = _chunkA1(s, acc)

            @pl.when(g + 2 < NCH1)
            def _():
                _issue1(g + 2, s, False)
        return acc
    acc = lax.fori_loop(0, NCH1 // 2, _outerA1, acc)

    def _chunkA2(s, acc):
        def bd(i, acc):
            accg, accl = acc
            off = pl.ds(pl.multiple_of(i * L, L), L)
            ra = idx_a[s, off]
            rb = idx_b[s, off]
            prd = idx_c[s, off]
            al = par_a[s, off]
            bt = par_b[s, off]
            gm = par_c[s, off]
            wg = plsc.load_gather(cnt_t, [prd])
            wl = (plsc.load_gather(cnt_t, [ra])
                  + plsc.load_gather(cnt_t, [rb]))
            ng, nl = [], []
            for b in range(BT):
                k = al * jnp.exp(bt * Lb[b] + gm * nit[b]) * den[b]
                ya = plsc.load_gather(y_t, [bspl[b], ra])
                yb = plsc.load_gather(y_t, [bspl[b], rb])
                r = k * ya * yb
                ng.append(accg[b] + wg * r)
                nl.append(accl[b] + wl * r)
                rn = -r
                plsc.addupdate_scatter(out_t, [bspl[b], prd], r)
                plsc.addupdate_scatter(out_t, [bspl[b], ra], rn)
                plsc.addupdate_scatter(out_t, [bspl[b], rb], rn)
            return (tuple(ng), tuple(nl))
        return lax.fori_loop(0, C // L, bd, acc)

    _issue2(0, 0)
    _issue2(1, 1)

    def _outerA2(h, acc):
        for s in range(2):
            g = h * 2 + s
            _drain2(g, s)
            acc = _chunkA2(s, acc)

            @pl.when(g + 2 < NCH2)
            def _():
                _issue2(g + 2, s)
        return acc
    acc = lax.fori_loop(0, NCH2 // 2, _outerA2, acc)

    accg, accl = acc
    ks = []
    for b in range(BT):
        gv = jnp.full((L,), jnp.sum(accg[b]), f32)
        lv = jnp.full((L,), jnp.sum(accl[b]), f32)
        ks.append(jnp.maximum(gv, 0.0) / (ym[b] + 1e-30)
                  + jnp.maximum(lv, 0.0) / (ys[b] + 1e-30))

    def _chunkB(s):
        def bd(i, carry):
            off = pl.ds(pl.multiple_of(i * L, L), L)
            reac = idx_a[s, off]
            prod = idx_b[s, off]
            al = par_a[s, off]
            bt = par_b[s, off]
            gm = par_c[s, off]
            phm = flg_a[s, off] > 0.0
            smm = flg_b[s, off] > 0.0
            for b in range(BT):
                k = al * jnp.exp(bt * Lb[b] + gm * nit[b])
                k = k * jnp.where(phm, dec[b], ones)
                k = jnp.where(smm, ks[b], k)
                yv = plsc.load_gather(y_t, [bspl[b], reac])
                r = k * yv
                plsc.addupdate_scatter(out_t, [bspl[b], prod], r)
                plsc.addupdate_scatter(out_t, [bspl[b], reac], -r)
            return carry
        lax.fori_loop(0, C // L, bd, 0)

    _issue1(0, 0, True)
    _issue1(1, 1, True)

    def _outerB(h, carry):
        for s in range(2):
            g = h * 2 + s
            _drain1(g, s, True)
            _chunkB(s)

            @pl.when(g + 2 < NCH1)
            def _():
                _issue1(g + 2, s, True)
        return carry
    lax.fori_loop(0, NCH1 // 2, _outerB, 0)

    pltpu.sync_copy(out_t, out_h.at[pl.ds(tid * BT, BT)])


_sc_call = pl.kernel(
    _sc_body,
    out_type=jax.ShapeDtypeStruct((B, N), f32),
    mesh=plsc.VectorSubcoreMesh(core_axis_name="c", subcore_axis_name="s"),
    scratch_types=[
        pltpu.VMEM((BT, N), f32),
        pltpu.VMEM((BT, N), f32),
        pltpu.VMEM((N,), f32),
        pltpu.VMEM((2, C), i32),
        pltpu.VMEM((2, C), i32),
        pltpu.VMEM((2, C), i32),
        pltpu.VMEM((2, C), f32),
        pltpu.VMEM((2, C), f32),
        pltpu.VMEM((2, C), f32),
        pltpu.VMEM((2, C), f32),
        pltpu.VMEM((2, C), f32),
        pltpu.VMEM((BT, 8), f32),
        pltpu.VMEM_SHARED((R1,), f32),
        pltpu.VMEM_SHARED((R1,), f32),
        pltpu.SemaphoreType.DMA,
        pltpu.SemaphoreType.DMA,
        pltpu.SemaphoreType.DMA,
    ],
)


def kernel(t_in, y_in, alpha, beta, gamma, reac_1st, prod_1st, reac_2nd_a,
           reac_2nd_b, prod_2nd, inds_surf, inds_mant, inds_id_photodeso,
           inds_id_smt):
    scal8 = pl.pallas_call(
        _med_body,
        out_shape=jax.ShapeDtypeStruct((8, B), f32),
    )(t_in.astype(f32).reshape(1, B))
    scal = scal8.T

    return _sc_call(
        y_in.astype(f32), scal,
        alpha.astype(f32), beta.astype(f32), gamma.astype(f32),
        reac_1st.astype(i32), prod_1st.astype(i32),
        reac_2nd_a.astype(i32), reac_2nd_b.astype(i32),
        prod_2nd.astype(i32),
        inds_surf.astype(i32), inds_mant.astype(i32),
        inds_id_photodeso.astype(i32), inds_id_smt.astype(i32),
    )


# speedup vs baseline: 7.1672x; 7.1672x over previous
import jax
import jax.numpy as jnp
from jax import lax
from jax.experimental import pallas as pl
from jax.experimental.pallas import tpu as pltpu
from jax.experimental.pallas import tpu_sc as plsc

B = 128
N = 10000
R1 = 80000
R2 = 80000
NSURF = 2000
NMANT = 2000
NPH = 1000
NSMT = 1000
AG = 100.0

NC = 2
NS = 16
NW = NC * NS
BT = B // NW
C = 1600
NCH1 = R1 // C
NCH2 = R2 // C
FSPAN = 3200
NFW = R1 // FSPAN
L = 16

f32 = jnp.float32
i32 = jnp.int32


def _med_body(t_ref, o_ref):
    t = t_ref[...]
    arg = t * 1e-5
    T = 10.0 + 5.0 * jnp.sin(arg)
    den = 1.0e4 * (1.0 + 0.1 * jnp.cos(arg))
    logT = jnp.log(T / 300.0)
    nit = -1.0 / T
    z = jnp.zeros_like(T)
    o_ref[...] = jnp.concatenate([logT, nit, den] + [z] * 13, axis=0)


def _flags_body(ph_h, smt_h, phf_h, smtf_h, idx_t, buf_t):
    cid = lax.axis_index("c")
    sid = lax.axis_index("s")
    tid = cid * NS + sid
    iot = lax.iota(i32, L)
    ones = jnp.ones((L,), f32)
    zf = jnp.zeros((L,), f32)

    @pl.when(tid < NFW)
    def _():
        base = tid * FSPAN

        def build(src_h, dst_h):
            pltpu.sync_copy(src_h, idx_t.at[pl.ds(0, NPH)])
            tail0 = (NPH // L) * L
            tv = idx_t[pl.ds(tail0, L)]
            tv = jnp.where(iot < (NPH - tail0), tv, jnp.full((L,), -1, i32))
            idx_t[pl.ds(tail0, L)] = tv

            def zb(i, carry):
                buf_t[pl.ds(pl.multiple_of(i * L, L), L)] = zf
                return carry
            lax.fori_loop(0, FSPAN // L, zb, 0)

            def bd(i, carry):
                idx = idx_t[pl.ds(pl.multiple_of(i * L, L), L)]
                loc = idx - base
                m = (loc >= 0) & (loc < FSPAN)
                locc = jnp.clip(loc, 0, FSPAN - 1)
                plsc.store_scatter(buf_t, [locc], ones, mask=m)
                return carry
            lax.fori_loop(0, (NPH + L - 1) // L, bd, 0)
            pltpu.sync_copy(buf_t, dst_h.at[pl.ds(pl.multiple_of(base, 8),
                                                  FSPAN)])

        build(ph_h, phf_h)
        build(smt_h, smtf_h)


_flags_call = pl.kernel(
    _flags_body,
    out_type=(jax.ShapeDtypeStruct((R1,), f32),
              jax.ShapeDtypeStruct((R1,), f32)),
    mesh=plsc.VectorSubcoreMesh(core_axis_name="c", subcore_axis_name="s"),
    compiler_params=pltpu.CompilerParams(needs_layout_passes=False),
    scratch_types=[
        pltpu.VMEM((1024,), i32),
        pltpu.VMEM((FSPAN,), f32),
    ],
)


def _safe_adds(groups):
    for idx, ref, pairs in groups:
        for ofs, val in pairs:
            plsc.addupdate_scatter(ref, [idx + ofs], val)


def _sc_body(y_h, scal_h, alpha_h, beta_h, gamma_h, reac1_h, prod1_h,
             r2a_h, r2b_h, prod2_h, surf_h, mant_h, phf_h, smtf_h,
             out_h,
             y_t, out_t, cnt_t, idx_a, idx_b, idx_c,
             par_a, par_b, par_c,
             flg_a, flg_b, scal_t, sem0, sem1):
    cid = lax.axis_index("c")
    sid = lax.axis_index("s")
    tid = cid * NS + sid
    ones = jnp.ones((L,), f32)
    zf = jnp.zeros((L,), f32)
    sems = (sem0, sem1)

    pltpu.sync_copy(y_h.at[pl.ds(tid * (BT * N), BT * N)], y_t)
    pltpu.sync_copy(scal_h.at[pl.ds(tid * (BT * L), BT * L)], scal_t)
    pltpu.sync_copy(surf_h, idx_b.at[pl.ds(0, NSURF)])
    pltpu.sync_copy(mant_h, idx_c.at[pl.ds(0, NMANT)])

    def _zero_range(ref, nsteps):
        def bd(i, carry):
            ref[pl.ds(pl.multiple_of(i * L, L), L)] = zf
            return carry
        lax.fori_loop(0, nsteps, bd, 0)

    _zero_range(out_t, BT * N // L)
    _zero_range(cnt_t, N // L)

    bofs = [jnp.full((L,), b * N, i32) for b in range(BT)]

    def _ssum(i, acc):
        idx = idx_b[pl.ds(pl.multiple_of(i * L, L), L)]
        plsc.addupdate_scatter(cnt_t, [idx], ones)
        return tuple(acc[b] + plsc.load_gather(y_t, [idx + bofs[b]])
                     for b in range(BT))
    acc_s = lax.fori_loop(0, NSURF // L, _ssum, (zf,) * BT)

    def _msum(i, acc):
        idx = idx_c[pl.ds(pl.multiple_of(i * L, L), L)]
        return tuple(acc[b] + plsc.load_gather(y_t, [idx + bofs[b]])
                     for b in range(BT))
    acc_m = lax.fori_loop(0, NMANT // L, _msum, (zf,) * BT)

    ys = [jnp.full((L,), jnp.sum(acc_s[b]), f32) for b in range(BT)]
    ym = [jnp.full((L,), jnp.sum(acc_m[b]), f32) for b in range(BT)]
    dec = [jnp.minimum(1.0 / (AG * (ys[b] + ym[b])), 1.0) for b in range(BT)]
    srow = [scal_t[pl.ds(b * L, L)] for b in range(BT)]
    Lb = [jnp.full((L,), srow[b][0], f32) for b in range(BT)]
    nit = [jnp.full((L,), srow[b][1], f32) for b in range(BT)]
    den = [jnp.full((L,), srow[b][2], f32) for b in range(BT)]

    def _issue1(g, s, with_smt):
        off = pl.ds(pl.multiple_of(g * C, 8), C)
        dst = pl.ds(s * C, C)
        sem = sems[s]
        pltpu.async_copy(reac1_h.at[off], idx_a.at[dst], sem)
        pltpu.async_copy(prod1_h.at[off], idx_b.at[dst], sem)
        pltpu.async_copy(alpha_h.at[off], par_a.at[dst], sem)
        pltpu.async_copy(beta_h.at[off], par_b.at[dst], sem)
        pltpu.async_copy(gamma_h.at[off], par_c.at[dst], sem)
        pltpu.async_copy(phf_h.at[off], flg_a.at[dst], sem)
        if with_smt:
            pltpu.async_copy(smtf_h.at[off], flg_b.at[dst], sem)

    def _drain1(g, s, with_smt):
        off = pl.ds(pl.multiple_of(g * C, 8), C)
        dst = pl.ds(s * C, C)
        sem = sems[s]
        pltpu.make_async_copy(reac1_h.at[off], idx_a.at[dst], sem).wait()
        pltpu.make_async_copy(prod1_h.at[off], idx_b.at[dst], sem).wait()
        pltpu.make_async_copy(alpha_h.at[off], par_a.at[dst], sem).wait()
        pltpu.make_async_copy(beta_h.at[off], par_b.at[dst], sem).wait()
        pltpu.make_async_copy(gamma_h.at[off], par_c.at[dst], sem).wait()
        pltpu.make_async_copy(phf_h.at[off], flg_a.at[dst], sem).wait()
        if with_smt:
            pltpu.make_async_copy(smtf_h.at[off], flg_b.at[dst], sem).wait()

    def _issue2(g, s):
        off = pl.ds(pl.multiple_of(g * C, 8), C)
        off2 = pl.ds(pl.multiple_of(R1 + g * C, 8), C)
        dst = pl.ds(s * C, C)
        sem = sems[s]
        pltpu.async_copy(r2a_h.at[off], idx_a.at[dst], sem)
        pltpu.async_copy(r2b_h.at[off], idx_b.at[dst], sem)
        pltpu.async_copy(prod2_h.at[off], idx_c.at[dst], sem)
        pltpu.async_copy(alpha_h.at[off2], par_a.at[dst], sem)
        pltpu.async_copy(beta_h.at[off2], par_b.at[dst], sem)
        pltpu.async_copy(gamma_h.at[off2], par_c.at[dst], sem)

    def _drain2(g, s):
        off = pl.ds(pl.multiple_of(g * C, 8), C)
        off2 = pl.ds(pl.multiple_of(R1 + g * C, 8), C)
        dst = pl.ds(s * C, C)
        sem = sems[s]
        pltpu.make_async_copy(r2a_h.at[off], idx_a.at[dst], sem).wait()
        pltpu.make_async_copy(r2b_h.at[off], idx_b.at[dst], sem).wait()
        pltpu.make_async_copy(prod2_h.at[off], idx_c.at[dst], sem).wait()
        pltpu.make_async_copy(alpha_h.at[off2], par_a.at[dst], sem).wait()
        pltpu.make_async_copy(beta_h.at[off2], par_b.at[dst], sem).wait()
        pltpu.make_async_copy(gamma_h.at[off2], par_c.at[dst], sem).wait()

    def _chunkA1(s, acc):
        def bd(i, acc):
            accg, accl = acc
            off = pl.ds(pl.multiple_of(s * C + i * L, L), L)
            reac = idx_a[off]
            prod = idx_b[off]
            al = par_a[off]
            bt = par_b[off]
            gm = par_c[off]
            phm = flg_a[off] > 0.0
            wg = plsc.load_gather(cnt_t, [prod])
            wl = plsc.load_gather(cnt_t, [reac])
            ng, nl = [], []
            for b in range(BT):
                k = al * jnp.exp(bt * Lb[b] + gm * nit[b])
                k = k * jnp.where(phm, dec[b], ones)
                yv = plsc.load_gather(y_t, [reac + bofs[b]])
                r = k * yv
                ng.append(accg[b] + wg * r)
                nl.append(accl[b] + wl * r)
            return (tuple(ng), tuple(nl))
        return lax.fori_loop(0, C // L, bd, acc)

    acc = ((zf,) * BT, (zf,) * BT)
    _issue1(0, 0, False)
    _issue1(1, 1, False)

    def _outerA1(h, acc):
        for s in range(2):
            g = h * 2 + s
            _drain1(g, s, False)
            acc = _chunkA1(s, acc)

            @pl.when(g + 2 < NCH1)
            def _():
                _issue1(g + 2, s, False)
        return acc
    acc = lax.fori_loop(0, NCH1 // 2, _outerA1, acc)

    def _chunkA2(s, acc):
        def bd(i, acc):
            accg, accl = acc
            off = pl.ds(pl.multiple_of(s * C + i * L, L), L)
            ra = idx_a[off]
            rb = idx_b[off]
            prd = idx_c[off]
            al = par_a[off]
            bt = par_b[off]
            gm = par_c[off]
            wg = plsc.load_gather(cnt_t, [prd])
            wl = (plsc.load_gather(cnt_t, [ra])
                  + plsc.load_gather(cnt_t, [rb]))
            ng, nl = [], []
            vp, va, vb = [], [], []
            for b in range(BT):
                k = al * jnp.exp(bt * Lb[b] + gm * nit[b]) * den[b]
                ya = plsc.load_gather(y_t, [ra + bofs[b]])
                yb = plsc.load_gather(y_t, [rb + bofs[b]])
                r = k * ya * yb
                ng.append(accg[b] + wg * r)
                nl.append(accl[b] + wl * r)
                rn = -r
                vp.append((bofs[b], r))
                va.append((bofs[b], rn))
                vb.append((bofs[b], rn))
            _safe_adds([(prd, out_t, vp), (ra, out_t, va),
                        (rb, out_t, vb)])
            return (tuple(ng), tuple(nl))
        return lax.fori_loop(0, C // L, bd, acc)

    _issue2(0, 0)
    _issue2(1, 1)

    def _outerA2(h, acc):
        for s in range(2):
            g = h * 2 + s
            _drain2(g, s)
            acc = _chunkA2(s, acc)

            @pl.when(g + 2 < NCH2)
            def _():
                _issue2(g + 2, s)
        return acc
    acc = lax.fori_loop(0, NCH2 // 2, _outerA2, acc)

    accg, accl = acc
    ks = []
    for b in range(BT):
        gv = jnp.full((L,), jnp.sum(accg[b]), f32)
        lv = jnp.full((L,), jnp.sum(accl[b]), f32)
        ks.append(jnp.maximum(gv, 0.0) / (ym[b] + 1e-30)
                  + jnp.maximum(lv, 0.0) / (ys[b] + 1e-30))

    def _chunkB(s):
        def bd(i, carry):
            off = pl.ds(pl.multiple_of(s * C + i * L, L), L)
            reac = idx_a[off]
            prod = idx_b[off]
            al = par_a[off]
            bt = par_b[off]
            gm = par_c[off]
            phm = flg_a[off] > 0.0
            smm = flg_b[off] > 0.0
            vp, vr = [], []
            for b in range(BT):
                k = al * jnp.exp(bt * Lb[b] + gm * nit[b])
                k = k * jnp.where(phm, dec[b], ones)
                k = jnp.where(smm, ks[b], k)
                yv = plsc.load_gather(y_t, [reac + bofs[b]])
                r = k * yv
                vp.append((bofs[b], r))
                vr.append((bofs[b], -r))
            _safe_adds([(prod, out_t, vp), (reac, out_t, vr)])
            return carry
        lax.fori_loop(0, C // L, bd, 0)

    _issue1(0, 0, True)
    _issue1(1, 1, True)

    def _outerB(h, carry):
        for s in range(2):
            g = h * 2 + s
            _drain1(g, s, True)
            _chunkB(s)

            @pl.when(g + 2 < NCH1)
            def _():
                _issue1(g + 2, s, True)
        return carry
    lax.fori_loop(0, NCH1 // 2, _outerB, 0)

    pltpu.sync_copy(out_t, out_h.at[pl.ds(tid * (BT * N), BT * N)])


_sc_call = pl.kernel(
    _sc_body,
    out_type=jax.ShapeDtypeStruct((B * N,), f32),
    mesh=plsc.VectorSubcoreMesh(core_axis_name="c", subcore_axis_name="s"),
    compiler_params=pltpu.CompilerParams(needs_layout_passes=False),
    scratch_types=[
        pltpu.VMEM((BT * N,), f32),
        pltpu.VMEM((BT * N,), f32),
        pltpu.VMEM((N,), f32),
        pltpu.VMEM((2 * C,), i32),
        pltpu.VMEM((2 * C,), i32),
        pltpu.VMEM((2 * C,), i32),
        pltpu.VMEM((2 * C,), f32),
        pltpu.VMEM((2 * C,), f32),
        pltpu.VMEM((2 * C,), f32),
        pltpu.VMEM((2 * C,), f32),
        pltpu.VMEM((2 * C,), f32),
        pltpu.VMEM((BT * L,), f32),
        pltpu.SemaphoreType.DMA,
        pltpu.SemaphoreType.DMA,
    ],
)


def kernel(t_in, y_in, alpha, beta, gamma, reac_1st, prod_1st, reac_2nd_a,
           reac_2nd_b, prod_2nd, inds_surf, inds_mant, inds_id_photodeso,
           inds_id_smt):
    scal16 = pl.pallas_call(
        _med_body,
        out_shape=jax.ShapeDtypeStruct((16, B), f32),
    )(t_in.astype(f32).reshape(1, B))
    scal = scal16.T.reshape(-1)

    ph_flags, smt_flags = _flags_call(
        inds_id_photodeso.astype(i32), inds_id_smt.astype(i32))

    out = _sc_call(
        y_in.astype(f32).reshape(-1), scal,
        alpha.astype(f32), beta.astype(f32), gamma.astype(f32),
        reac_1st.astype(i32), prod_1st.astype(i32),
        reac_2nd_a.astype(i32), reac_2nd_b.astype(i32),
        prod_2nd.astype(i32),
        inds_surf.astype(i32), inds_mant.astype(i32),
        ph_flags, smt_flags,
    )
    return out.reshape(B, N)


# speedup vs baseline: 8.4126x; 1.1738x over previous
import jax
import jax.numpy as jnp
from jax import lax
from jax.experimental import pallas as pl
from jax.experimental.pallas import tpu as pltpu
from jax.experimental.pallas import tpu_sc as plsc

B = 128
N = 10000
R1 = 80000
R2 = 80000
NSURF = 2000
NMANT = 2000
NPH = 1000
NSMT = 1000
AG = 100.0

NC = 2
NS = 16
NW = NC * NS
BT = B // NW
C = 1600
NCH1 = R1 // C
NCH2 = R2 // C
FSPAN = 3200
NFW = R1 // FSPAN
L = 16

f32 = jnp.float32
i32 = jnp.int32


def _med_body(t_ref, o_ref):
    t = t_ref[...]
    arg = t * 1e-5
    T = 10.0 + 5.0 * jnp.sin(arg)
    den = 1.0e4 * (1.0 + 0.1 * jnp.cos(arg))
    logT = jnp.log(T / 300.0)
    nit = -1.0 / T
    z = jnp.zeros_like(T)
    o_ref[...] = jnp.concatenate([logT, nit, den] + [z] * 13, axis=0)


def _flags_body(ph_h, smt_h, phf_h, smtf_h, idx_t, buf_t):
    cid = lax.axis_index("c")
    sid = lax.axis_index("s")
    tid = cid * NS + sid
    iot = lax.iota(i32, L)
    ones = jnp.ones((L,), f32)
    zf = jnp.zeros((L,), f32)

    @pl.when(tid < NFW)
    def _():
        base = tid * FSPAN

        def build(src_h, dst_h):
            pltpu.sync_copy(src_h, idx_t.at[pl.ds(0, NPH)])
            tail0 = (NPH // L) * L
            tv = idx_t[pl.ds(tail0, L)]
            tv = jnp.where(iot < (NPH - tail0), tv, jnp.full((L,), -1, i32))
            idx_t[pl.ds(tail0, L)] = tv

            def zb(i, carry):
                buf_t[pl.ds(pl.multiple_of(i * L, L), L)] = zf
                return carry
            lax.fori_loop(0, FSPAN // L, zb, 0)

            def bd(i, carry):
                idx = idx_t[pl.ds(pl.multiple_of(i * L, L), L)]
                loc = idx - base
                m = (loc >= 0) & (loc < FSPAN)
                locc = jnp.clip(loc, 0, FSPAN - 1)
                plsc.store_scatter(buf_t, [locc], ones, mask=m)
                return carry
            lax.fori_loop(0, (NPH + L - 1) // L, bd, 0)
            pltpu.sync_copy(buf_t, dst_h.at[pl.ds(pl.multiple_of(base, 8),
                                                  FSPAN)])

        build(ph_h, phf_h)
        build(smt_h, smtf_h)


_flags_call = pl.kernel(
    _flags_body,
    out_type=(jax.ShapeDtypeStruct((R1,), f32),
              jax.ShapeDtypeStruct((R1,), f32)),
    mesh=plsc.VectorSubcoreMesh(core_axis_name="c", subcore_axis_name="s"),
    compiler_params=pltpu.CompilerParams(needs_layout_passes=False),
    scratch_types=[
        pltpu.VMEM((1024,), i32),
        pltpu.VMEM((FSPAN,), f32),
    ],
)


def _safe_adds(groups):
    for idx, ref, pairs in groups:
        for ofs, val in pairs:
            plsc.addupdate_scatter(ref, [idx + ofs], val)


def _sc_body(y_h, scal_h, alpha_h, beta_h, gamma_h, reac1_h, prod1_h,
             r2a_h, r2b_h, prod2_h, surf_h, mant_h, phf_h, smtf_h,
             out_h,
             y_t, out_t, cnt_t, idx_a, idx_b, idx_c,
             par_a, par_b, par_c,
             flg_a, flg_b, scal_t, smt_t, sem0, sem1):
    cid = lax.axis_index("c")
    sid = lax.axis_index("s")
    tid = cid * NS + sid
    ones = jnp.ones((L,), f32)
    zf = jnp.zeros((L,), f32)
    sems = (sem0, sem1)

    pltpu.sync_copy(y_h.at[pl.ds(tid * (BT * N), BT * N)], y_t)
    pltpu.sync_copy(scal_h.at[pl.ds(tid * (BT * L), BT * L)], scal_t)
    pltpu.sync_copy(surf_h, idx_b.at[pl.ds(0, NSURF)])
    pltpu.sync_copy(mant_h, idx_c.at[pl.ds(0, NMANT)])

    def _zero_range(ref, nsteps):
        def bd(i, carry):
            ref[pl.ds(pl.multiple_of(i * L, L), L)] = zf
            return carry
        lax.fori_loop(0, nsteps, bd, 0)

    _zero_range(out_t, BT * N // L)
    _zero_range(cnt_t, N // L)

    neg = jnp.full((L,), -1, i32)

    def _sent(i, carry):
        smt_t[pl.ds(pl.multiple_of(i * L, L), L)] = neg
        return carry
    lax.fori_loop(0, 1024 // L, _sent, 0)

    bofs = [jnp.full((L,), b * N, i32) for b in range(BT)]

    def _ssum(i, acc):
        idx = idx_b[pl.ds(pl.multiple_of(i * L, L), L)]
        plsc.addupdate_scatter(cnt_t, [idx], ones)
        return tuple(acc[b] + plsc.load_gather(y_t, [idx + bofs[b]])
                     for b in range(BT))
    acc_s = lax.fori_loop(0, NSURF // L, _ssum, (zf,) * BT)

    def _msum(i, acc):
        idx = idx_c[pl.ds(pl.multiple_of(i * L, L), L)]
        return tuple(acc[b] + plsc.load_gather(y_t, [idx + bofs[b]])
                     for b in range(BT))
    acc_m = lax.fori_loop(0, NMANT // L, _msum, (zf,) * BT)

    ys = [jnp.full((L,), jnp.sum(acc_s[b]), f32) for b in range(BT)]
    ym = [jnp.full((L,), jnp.sum(acc_m[b]), f32) for b in range(BT)]
    dec = [jnp.minimum(1.0 / (AG * (ys[b] + ym[b])), 1.0) for b in range(BT)]
    srow = [scal_t[pl.ds(b * L, L)] for b in range(BT)]
    Lb = [jnp.full((L,), srow[b][0], f32) for b in range(BT)]
    nit = [jnp.full((L,), srow[b][1], f32) for b in range(BT)]
    den = [jnp.full((L,), srow[b][2], f32) for b in range(BT)]

    def _issue1(g, s):
        off = pl.ds(pl.multiple_of(g * C, 8), C)
        dst = pl.ds(s * C, C)
        sem = sems[s]
        pltpu.async_copy(reac1_h.at[off], idx_a.at[dst], sem)
        pltpu.async_copy(prod1_h.at[off], idx_b.at[dst], sem)
        pltpu.async_copy(alpha_h.at[off], par_a.at[dst], sem)
        pltpu.async_copy(beta_h.at[off], par_b.at[dst], sem)
        pltpu.async_copy(gamma_h.at[off], par_c.at[dst], sem)
        pltpu.async_copy(phf_h.at[off], flg_a.at[dst], sem)
        pltpu.async_copy(smtf_h.at[off], flg_b.at[dst], sem)

    def _drain1(g, s):
        off = pl.ds(pl.multiple_of(g * C, 8), C)
        dst = pl.ds(s * C, C)
        sem = sems[s]
        pltpu.make_async_copy(reac1_h.at[off], idx_a.at[dst], sem).wait()
        pltpu.make_async_copy(prod1_h.at[off], idx_b.at[dst], sem).wait()
        pltpu.make_async_copy(alpha_h.at[off], par_a.at[dst], sem).wait()
        pltpu.make_async_copy(beta_h.at[off], par_b.at[dst], sem).wait()
        pltpu.make_async_copy(gamma_h.at[off], par_c.at[dst], sem).wait()
        pltpu.make_async_copy(phf_h.at[off], flg_a.at[dst], sem).wait()
        pltpu.make_async_copy(smtf_h.at[off], flg_b.at[dst], sem).wait()

    def _issue2(g, s):
        off = pl.ds(pl.multiple_of(g * C, 8), C)
        off2 = pl.ds(pl.multiple_of(R1 + g * C, 8), C)
        dst = pl.ds(s * C, C)
        sem = sems[s]
        pltpu.async_copy(r2a_h.at[off], idx_a.at[dst], sem)
        pltpu.async_copy(r2b_h.at[off], idx_b.at[dst], sem)
        pltpu.async_copy(prod2_h.at[off], idx_c.at[dst], sem)
        pltpu.async_copy(alpha_h.at[off2], par_a.at[dst], sem)
        pltpu.async_copy(beta_h.at[off2], par_b.at[dst], sem)
        pltpu.async_copy(gamma_h.at[off2], par_c.at[dst], sem)

    def _drain2(g, s):
        off = pl.ds(pl.multiple_of(g * C, 8), C)
        off2 = pl.ds(pl.multiple_of(R1 + g * C, 8), C)
        dst = pl.ds(s * C, C)
        sem = sems[s]
        pltpu.make_async_copy(r2a_h.at[off], idx_a.at[dst], sem).wait()
        pltpu.make_async_copy(r2b_h.at[off], idx_b.at[dst], sem).wait()
        pltpu.make_async_copy(prod2_h.at[off], idx_c.at[dst], sem).wait()
        pltpu.make_async_copy(alpha_h.at[off2], par_a.at[dst], sem).wait()
        pltpu.make_async_copy(beta_h.at[off2], par_b.at[dst], sem).wait()
        pltpu.make_async_copy(gamma_h.at[off2], par_c.at[dst], sem).wait()

    def _chunkA1(s, acc):
        def bd(i, acc):
            accg, accl, cap = acc
            off = pl.ds(pl.multiple_of(s * C + i * L, L), L)
            reac = idx_a[off]
            prod = idx_b[off]
            al = par_a[off]
            bt = par_b[off]
            gm = par_c[off]
            phm = flg_a[off] > 0.0
            smm = flg_b[off] > 0.0
            keep = jnp.logical_not(smm)
            wg = plsc.load_gather(cnt_t, [prod])
            wl = plsc.load_gather(cnt_t, [reac])
            ng, nl = [], []
            vp, vr = [], []
            for b in range(BT):
                k = al * jnp.exp(bt * Lb[b] + gm * nit[b])
                k = k * jnp.where(phm, dec[b], ones)
                yv = plsc.load_gather(y_t, [reac + bofs[b]])
                r = k * yv
                ng.append(accg[b] + wg * r)
                nl.append(accl[b] + wl * r)
                vp.append((bofs[b], r))
                vr.append((bofs[b], -r))
            for bo, val in vp:
                plsc.addupdate_scatter(out_t, [prod + bo], val, mask=keep)
            for bo, val in vr:
                plsc.addupdate_scatter(out_t, [reac + bo], val, mask=keep)
            smi = smm.astype(i32)
            cs = plsc.cumsum(smi)
            pos = jnp.clip(cap + cs - 1, 0, 1023)
            packed = reac * 16384 + prod
            plsc.store_scatter(smt_t, [pos], packed, mask=smm)
            pc = plsc.all_reduce_population_count(smm)
            return (tuple(ng), tuple(nl), cap + pc[0])
        return lax.fori_loop(0, C // L, bd, acc)

    acc = ((zf,) * BT, (zf,) * BT, jnp.int32(0))
    _issue1(0, 0)
    _issue1(1, 1)

    def _outerA1(h, acc):
        for s in range(2):
            g = h * 2 + s
            _drain1(g, s)
            acc = _chunkA1(s, acc)

            @pl.when(g + 2 < NCH1)
            def _():
                _issue1(g + 2, s)
        return acc
    acc = lax.fori_loop(0, NCH1 // 2, _outerA1, acc)
    accg1, accl1, _cap = acc
    acc = (accg1, accl1)

    def _chunkA2(s, acc):
        def bd(i, acc):
            accg, accl = acc
            off = pl.ds(pl.multiple_of(s * C + i * L, L), L)
            ra = idx_a[off]
            rb = idx_b[off]
            prd = idx_c[off]
            al = par_a[off]
            bt = par_b[off]
            gm = par_c[off]
            wg = plsc.load_gather(cnt_t, [prd])
            wl = (plsc.load_gather(cnt_t, [ra])
                  + plsc.load_gather(cnt_t, [rb]))
            ng, nl = [], []
            vp, va, vb = [], [], []
            for b in range(BT):
                k = al * jnp.exp(bt * Lb[b] + gm * nit[b]) * den[b]
                ya = plsc.load_gather(y_t, [ra + bofs[b]])
                yb = plsc.load_gather(y_t, [rb + bofs[b]])
                r = k * ya * yb
                ng.append(accg[b] + wg * r)
                nl.append(accl[b] + wl * r)
                rn = -r
                vp.append((bofs[b], r))
                va.append((bofs[b], rn))
                vb.append((bofs[b], rn))
            _safe_adds([(prd, out_t, vp), (ra, out_t, va),
                        (rb, out_t, vb)])
            return (tuple(ng), tuple(nl))
        return lax.fori_loop(0, C // L, bd, acc)

    _issue2(0, 0)
    _issue2(1, 1)

    def _outerA2(h, acc):
        for s in range(2):
            g = h * 2 + s
            _drain2(g, s)
            acc = _chunkA2(s, acc)

            @pl.when(g + 2 < NCH2)
            def _():
                _issue2(g + 2, s)
        return acc
    acc = lax.fori_loop(0, NCH2 // 2, _outerA2, acc)

    accg, accl = acc
    ks = []
    for b in range(BT):
        gv = jnp.full((L,), jnp.sum(accg[b]), f32)
        lv = jnp.full((L,), jnp.sum(accl[b]), f32)
        ks.append(jnp.maximum(gv, 0.0) / (ym[b] + 1e-30)
                  + jnp.maximum(lv, 0.0) / (ys[b] + 1e-30))

    def _fix(i, carry):
        v = smt_t[pl.ds(pl.multiple_of(i * L, L), L)]
        m = v >= 0
        vv = jnp.where(m, v, 0)
        reac = jnp.right_shift(vv, 14)
        prod = jnp.bitwise_and(vv, 16383)
        for b in range(BT):
            yv = plsc.load_gather(y_t, [reac + bofs[b]])
            r = ks[b] * yv
            plsc.addupdate_scatter(out_t, [prod + bofs[b]], r, mask=m)
            plsc.addupdate_scatter(out_t, [reac + bofs[b]], -r, mask=m)
        return carry
    lax.fori_loop(0, 1024 // L, _fix, 0)

    pltpu.sync_copy(out_t, out_h.at[pl.ds(tid * (BT * N), BT * N)])


_sc_call = pl.kernel(
    _sc_body,
    out_type=jax.ShapeDtypeStruct((B * N,), f32),
    mesh=plsc.VectorSubcoreMesh(core_axis_name="c", subcore_axis_name="s"),
    compiler_params=pltpu.CompilerParams(needs_layout_passes=False),
    scratch_types=[
        pltpu.VMEM((BT * N,), f32),
        pltpu.VMEM((BT * N,), f32),
        pltpu.VMEM((N,), f32),
        pltpu.VMEM((2 * C,), i32),
        pltpu.VMEM((2 * C,), i32),
        pltpu.VMEM((2 * C,), i32),
        pltpu.VMEM((2 * C,), f32),
        pltpu.VMEM((2 * C,), f32),
        pltpu.VMEM((2 * C,), f32),
        pltpu.VMEM((2 * C,), f32),
        pltpu.VMEM((2 * C,), f32),
        pltpu.VMEM((BT * L,), f32),
        pltpu.VMEM((1024,), i32),
        pltpu.SemaphoreType.DMA,
        pltpu.SemaphoreType.DMA,
    ],
)


def kernel(t_in, y_in, alpha, beta, gamma, reac_1st, prod_1st, reac_2nd_a,
           reac_2nd_b, prod_2nd, inds_surf, inds_mant, inds_id_photodeso,
           inds_id_smt):
    scal16 = pl.pallas_call(
        _med_body,
        out_shape=jax.ShapeDtypeStruct((16, B), f32),
    )(t_in.astype(f32).reshape(1, B))
    scal = scal16.T.reshape(-1)

    ph_flags, smt_flags = _flags_call(
        inds_id_photodeso.astype(i32), inds_id_smt.astype(i32))

    out = _sc_call(
        y_in.astype(f32).reshape(-1), scal,
        alpha.astype(f32), beta.astype(f32), gamma.astype(f32),
        reac_1st.astype(i32), prod_1st.astype(i32),
        reac_2nd_a.astype(i32), reac_2nd_b.astype(i32),
        prod_2nd.astype(i32),
        inds_surf.astype(i32), inds_mant.astype(i32),
        ph_flags, smt_flags,
    )
    return out.reshape(B, N)


# speedup vs baseline: 10.3821x; 1.2341x over previous
import jax
import jax.numpy as jnp
from jax import lax
from jax.experimental import pallas as pl
from jax.experimental.pallas import tpu as pltpu
from jax.experimental.pallas import tpu_sc as plsc

B = 128
N = 10000
R1 = 80000
R2 = 80000
NSURF = 2000
NMANT = 2000
NPH = 1000
NSMT = 1000
AG = 100.0

NC = 2
NS = 16
NW = NC * NS
BT = B // NW
C = 1600
NCH1 = R1 // C
NCH2 = R2 // C
FSPAN = 3200
NFW = R1 // FSPAN
L = 16

f32 = jnp.float32
i32 = jnp.int32


def _med_body(t_ref, o_ref):
    t = t_ref[...]
    arg = t * 1e-5
    T = 10.0 + 5.0 * jnp.sin(arg)
    den = 1.0e4 * (1.0 + 0.1 * jnp.cos(arg))
    logT = jnp.log(T / 300.0)
    nit = -1.0 / T
    z = jnp.zeros_like(T)
    o_ref[...] = jnp.concatenate([logT, nit, den] + [z] * 13, axis=0)


def _flags_body(ph_h, smt_h, phf_h, smtf_h, idx_t, buf_t):
    cid = lax.axis_index("c")
    sid = lax.axis_index("s")
    tid = cid * NS + sid
    iot = lax.iota(i32, L)
    ones = jnp.ones((L,), f32)
    zf = jnp.zeros((L,), f32)

    @pl.when(tid < NFW)
    def _():
        base = tid * FSPAN

        def build(src_h, dst_h):
            pltpu.sync_copy(src_h, idx_t.at[pl.ds(0, NPH)])
            tail0 = (NPH // L) * L
            tv = idx_t[pl.ds(tail0, L)]
            tv = jnp.where(iot < (NPH - tail0), tv, jnp.full((L,), -1, i32))
            idx_t[pl.ds(tail0, L)] = tv

            def zb(i, carry):
                buf_t[pl.ds(pl.multiple_of(i * L, L), L)] = zf
                return carry
            lax.fori_loop(0, FSPAN // L, zb, 0)

            def bd(i, carry):
                idx = idx_t[pl.ds(pl.multiple_of(i * L, L), L)]
                loc = idx - base
                m = (loc >= 0) & (loc < FSPAN)
                locc = jnp.clip(loc, 0, FSPAN - 1)
                plsc.store_scatter(buf_t, [locc], ones, mask=m)
                return carry
            lax.fori_loop(0, (NPH + L - 1) // L, bd, 0)
            pltpu.sync_copy(buf_t, dst_h.at[pl.ds(pl.multiple_of(base, 8),
                                                  FSPAN)])

        build(ph_h, phf_h)
        build(smt_h, smtf_h)


_flags_call = pl.kernel(
    _flags_body,
    out_type=(jax.ShapeDtypeStruct((R1,), f32),
              jax.ShapeDtypeStruct((R1,), f32)),
    mesh=plsc.VectorSubcoreMesh(core_axis_name="c", subcore_axis_name="s"),
    compiler_params=pltpu.CompilerParams(needs_layout_passes=False),
    scratch_types=[
        pltpu.VMEM((1024,), i32),
        pltpu.VMEM((FSPAN,), f32),
    ],
)


def _safe_adds(groups):
    for idx, ref, pairs in groups:
        for ofs, val in pairs:
            plsc.addupdate_scatter(ref, [idx + ofs], val)


def _sc_body(y_h, scal_h, alpha_h, beta_h, gamma_h, reac1_h, prod1_h,
             r2a_h, r2b_h, prod2_h, surf_h, mant_h, phf_h, smtf_h,
             out_h,
             y_t, out_t, cnt_t, idx_a, idx_b, idx_c,
             par_a, par_b, par_c,
             flg_a, flg_b, scal_t, smt_t, sem0, sem1):
    cid = lax.axis_index("c")
    sid = lax.axis_index("s")
    tid = cid * NS + sid
    ones = jnp.ones((L,), f32)
    zf = jnp.zeros((L,), f32)
    sems = (sem0, sem1)

    pltpu.sync_copy(y_h.at[pl.ds(tid * (BT * N), BT * N)], y_t)
    pltpu.sync_copy(scal_h.at[pl.ds(tid * (BT * L), BT * L)], scal_t)
    pltpu.sync_copy(surf_h, idx_b.at[pl.ds(0, NSURF)])
    pltpu.sync_copy(mant_h, idx_c.at[pl.ds(0, NMANT)])

    def _zero_range(ref, nsteps):
        def bd(i, carry):
            ref[pl.ds(pl.multiple_of(i * L, L), L)] = zf
            return carry
        lax.fori_loop(0, nsteps, bd, 0)

    _zero_range(out_t, BT * N // L)
    _zero_range(cnt_t, N // L)

    neg = jnp.full((L,), -1, i32)

    def _sent(i, carry):
        smt_t[pl.ds(pl.multiple_of(i * L, L), L)] = neg
        return carry
    lax.fori_loop(0, 1024 // L, _sent, 0)

    bofs = [jnp.full((L,), b * N, i32) for b in range(BT)]

    def _ssum(i, acc):
        idx = idx_b[pl.ds(pl.multiple_of(i * L, L), L)]
        plsc.addupdate_scatter(cnt_t, [idx], ones)
        return tuple(acc[b] + plsc.load_gather(y_t, [idx + bofs[b]])
                     for b in range(BT))
    acc_s = lax.fori_loop(0, NSURF // L, _ssum, (zf,) * BT)

    def _msum(i, acc):
        idx = idx_c[pl.ds(pl.multiple_of(i * L, L), L)]
        return tuple(acc[b] + plsc.load_gather(y_t, [idx + bofs[b]])
                     for b in range(BT))
    acc_m = lax.fori_loop(0, NMANT // L, _msum, (zf,) * BT)

    ys = [jnp.full((L,), jnp.sum(acc_s[b]), f32) for b in range(BT)]
    ym = [jnp.full((L,), jnp.sum(acc_m[b]), f32) for b in range(BT)]
    dec = [jnp.minimum(1.0 / (AG * (ys[b] + ym[b])), 1.0) for b in range(BT)]
    srow = [scal_t[pl.ds(b * L, L)] for b in range(BT)]
    Lb = [jnp.full((L,), srow[b][0], f32) for b in range(BT)]
    nit = [jnp.full((L,), srow[b][1], f32) for b in range(BT)]
    den = [jnp.full((L,), srow[b][2], f32) for b in range(BT)]

    def _issue1(g, s):
        off = pl.ds(pl.multiple_of(g * C, 8), C)
        dst = pl.ds(s * C, C)
        sem = sems[s]
        pltpu.async_copy(reac1_h.at[off], idx_a.at[dst], sem)
        pltpu.async_copy(prod1_h.at[off], idx_b.at[dst], sem)
        pltpu.async_copy(alpha_h.at[off], par_a.at[dst], sem)
        pltpu.async_copy(beta_h.at[off], par_b.at[dst], sem)
        pltpu.async_copy(gamma_h.at[off], par_c.at[dst], sem)
        pltpu.async_copy(phf_h.at[off], flg_a.at[dst], sem)
        pltpu.async_copy(smtf_h.at[off], flg_b.at[dst], sem)

    def _drain1(g, s):
        off = pl.ds(pl.multiple_of(g * C, 8), C)
        dst = pl.ds(s * C, C)
        sem = sems[s]
        pltpu.make_async_copy(reac1_h.at[off], idx_a.at[dst], sem).wait()
        pltpu.make_async_copy(prod1_h.at[off], idx_b.at[dst], sem).wait()
        pltpu.make_async_copy(alpha_h.at[off], par_a.at[dst], sem).wait()
        pltpu.make_async_copy(beta_h.at[off], par_b.at[dst], sem).wait()
        pltpu.make_async_copy(gamma_h.at[off], par_c.at[dst], sem).wait()
        pltpu.make_async_copy(phf_h.at[off], flg_a.at[dst], sem).wait()
        pltpu.make_async_copy(smtf_h.at[off], flg_b.at[dst], sem).wait()

    def _issue2(g, s):
        off = pl.ds(pl.multiple_of(g * C, 8), C)
        off2 = pl.ds(pl.multiple_of(R1 + g * C, 8), C)
        dst = pl.ds(s * C, C)
        sem = sems[s]
        pltpu.async_copy(r2a_h.at[off], idx_a.at[dst], sem)
        pltpu.async_copy(r2b_h.at[off], idx_b.at[dst], sem)
        pltpu.async_copy(prod2_h.at[off], idx_c.at[dst], sem)
        pltpu.async_copy(alpha_h.at[off2], par_a.at[dst], sem)
        pltpu.async_copy(beta_h.at[off2], par_b.at[dst], sem)
        pltpu.async_copy(gamma_h.at[off2], par_c.at[dst], sem)

    def _drain2(g, s):
        off = pl.ds(pl.multiple_of(g * C, 8), C)
        off2 = pl.ds(pl.multiple_of(R1 + g * C, 8), C)
        dst = pl.ds(s * C, C)
        sem = sems[s]
        pltpu.make_async_copy(r2a_h.at[off], idx_a.at[dst], sem).wait()
        pltpu.make_async_copy(r2b_h.at[off], idx_b.at[dst], sem).wait()
        pltpu.make_async_copy(prod2_h.at[off], idx_c.at[dst], sem).wait()
        pltpu.make_async_copy(alpha_h.at[off2], par_a.at[dst], sem).wait()
        pltpu.make_async_copy(beta_h.at[off2], par_b.at[dst], sem).wait()
        pltpu.make_async_copy(gamma_h.at[off2], par_c.at[dst], sem).wait()

    def _chunkA1(s, acc):
        def bd(i, acc):
            accg, accl, cap = acc
            off = pl.ds(pl.multiple_of(s * C + i * L, L), L)
            reac = idx_a[off]
            prod = idx_b[off]
            al = par_a[off]
            bt = par_b[off]
            gm = par_c[off]
            phm = flg_a[off] > 0.0
            smm = flg_b[off] > 0.0
            keep = jnp.logical_not(smm)
            wg = plsc.load_gather(cnt_t, [prod])
            wl = plsc.load_gather(cnt_t, [reac])
            ng, nl = [], []
            vp, vr = [], []
            for b in range(BT):
                k = al * jnp.exp(bt * Lb[b] + gm * nit[b])
                k = k * jnp.where(phm, dec[b], ones)
                yv = plsc.load_gather(y_t, [reac + bofs[b]])
                r = k * yv
                ng.append(accg[b] + wg * r)
                nl.append(accl[b] + wl * r)
                vp.append((bofs[b], r))
                vr.append((bofs[b], -r))
            for bo, val in vp:
                plsc.addupdate_scatter(out_t, [prod + bo], val, mask=keep)
            for bo, val in vr:
                plsc.addupdate_scatter(out_t, [reac + bo], val, mask=keep)
            smi = smm.astype(i32)
            cs = plsc.cumsum(smi)
            pos = jnp.clip(cap + cs - 1, 0, 1023)
            packed = reac * 16384 + prod
            plsc.store_scatter(smt_t, [pos], packed, mask=smm)
            pc = plsc.all_reduce_population_count(smm)
            return (tuple(ng), tuple(nl), cap + pc[0])
        return plsc.parallel_loop(0, C // L, carry=acc)(bd)

    acc = ((zf,) * BT, (zf,) * BT, jnp.int32(0))
    _issue1(0, 0)
    _issue1(1, 1)

    def _outerA1(h, acc):
        for s in range(2):
            g = h * 2 + s
            _drain1(g, s)
            acc = _chunkA1(s, acc)

            @pl.when(g + 2 < NCH1)
            def _():
                _issue1(g + 2, s)
        return acc
    acc = lax.fori_loop(0, NCH1 // 2, _outerA1, acc)
    accg1, accl1, _cap = acc
    acc = (accg1, accl1)

    def _chunkA2(s, acc):
        def bd(i, acc):
            accg, accl = acc
            off = pl.ds(pl.multiple_of(s * C + i * L, L), L)
            ra = idx_a[off]
            rb = idx_b[off]
            prd = idx_c[off]
            al = par_a[off]
            bt = par_b[off]
            gm = par_c[off]
            wg = plsc.load_gather(cnt_t, [prd])
            wl = (plsc.load_gather(cnt_t, [ra])
                  + plsc.load_gather(cnt_t, [rb]))
            ng, nl = [], []
            vp, va, vb = [], [], []
            for b in range(BT):
                k = al * jnp.exp(bt * Lb[b] + gm * nit[b]) * den[b]
                ya = plsc.load_gather(y_t, [ra + bofs[b]])
                yb = plsc.load_gather(y_t, [rb + bofs[b]])
                r = k * ya * yb
                ng.append(accg[b] + wg * r)
                nl.append(accl[b] + wl * r)
                rn = -r
                vp.append((bofs[b], r))
                va.append((bofs[b], rn))
                vb.append((bofs[b], rn))
            _safe_adds([(prd, out_t, vp), (ra, out_t, va),
                        (rb, out_t, vb)])
            return (tuple(ng), tuple(nl))
        return plsc.parallel_loop(0, C // L, carry=acc)(bd)

    _issue2(0, 0)
    _issue2(1, 1)

    def _outerA2(h, acc):
        for s in range(2):
            g = h * 2 + s
            _drain2(g, s)
            acc = _chunkA2(s, acc)

            @pl.when(g + 2 < NCH2)
            def _():
                _issue2(g + 2, s)
        return acc
    acc = lax.fori_loop(0, NCH2 // 2, _outerA2, acc)

    accg, accl = acc
    ks = []
    for b in range(BT):
        gv = jnp.full((L,), jnp.sum(accg[b]), f32)
        lv = jnp.full((L,), jnp.sum(accl[b]), f32)
        ks.append(jnp.maximum(gv, 0.0) / (ym[b] + 1e-30)
                  + jnp.maximum(lv, 0.0) / (ys[b] + 1e-30))

    def _fix(i, carry):
        v = smt_t[pl.ds(pl.multiple_of(i * L, L), L)]
        m = v >= 0
        vv = jnp.where(m, v, 0)
        reac = jnp.right_shift(vv, 14)
        prod = jnp.bitwise_and(vv, 16383)
        for b in range(BT):
            yv = plsc.load_gather(y_t, [reac + bofs[b]])
            r = ks[b] * yv
            plsc.addupdate_scatter(out_t, [prod + bofs[b]], r, mask=m)
            plsc.addupdate_scatter(out_t, [reac + bofs[b]], -r, mask=m)
        return carry
    lax.fori_loop(0, 1024 // L, _fix, 0)

    pltpu.sync_copy(out_t, out_h.at[pl.ds(tid * (BT * N), BT * N)])


_sc_call = pl.kernel(
    _sc_body,
    out_type=jax.ShapeDtypeStruct((B * N,), f32),
    mesh=plsc.VectorSubcoreMesh(core_axis_name="c", subcore_axis_name="s"),
    compiler_params=pltpu.CompilerParams(needs_layout_passes=False),
    scratch_types=[
        pltpu.VMEM((BT * N,), f32),
        pltpu.VMEM((BT * N,), f32),
        pltpu.VMEM((N,), f32),
        pltpu.VMEM((2 * C,), i32),
        pltpu.VMEM((2 * C,), i32),
        pltpu.VMEM((2 * C,), i32),
        pltpu.VMEM((2 * C,), f32),
        pltpu.VMEM((2 * C,), f32),
        pltpu.VMEM((2 * C,), f32),
        pltpu.VMEM((2 * C,), f32),
        pltpu.VMEM((2 * C,), f32),
        pltpu.VMEM((BT * L,), f32),
        pltpu.VMEM((1024,), i32),
        pltpu.SemaphoreType.DMA,
        pltpu.SemaphoreType.DMA,
    ],
)


def kernel(t_in, y_in, alpha, beta, gamma, reac_1st, prod_1st, reac_2nd_a,
           reac_2nd_b, prod_2nd, inds_surf, inds_mant, inds_id_photodeso,
           inds_id_smt):
    scal16 = pl.pallas_call(
        _med_body,
        out_shape=jax.ShapeDtypeStruct((16, B), f32),
    )(t_in.astype(f32).reshape(1, B))
    scal = scal16.T.reshape(-1)

    ph_flags, smt_flags = _flags_call(
        inds_id_photodeso.astype(i32), inds_id_smt.astype(i32))

    out = _sc_call(
        y_in.astype(f32).reshape(-1), scal,
        alpha.astype(f32), beta.astype(f32), gamma.astype(f32),
        reac_1st.astype(i32), prod_1st.astype(i32),
        reac_2nd_a.astype(i32), reac_2nd_b.astype(i32),
        prod_2nd.astype(i32),
        inds_surf.astype(i32), inds_mant.astype(i32),
        ph_flags, smt_flags,
    )
    return out.reshape(B, N)


# speedup vs baseline: 10.4453x; 1.0061x over previous
import jax
import jax.numpy as jnp
from jax import lax
from jax.experimental import pallas as pl
from jax.experimental.pallas import tpu as pltpu
from jax.experimental.pallas import tpu_sc as plsc

B = 128
N = 10000
R1 = 80000
R2 = 80000
NSURF = 2000
NMANT = 2000
NPH = 1000
NSMT = 1000
AG = 100.0

NC = 2
NS = 16
NW = NC * NS
BT = B // NW
C = 1600
NCH1 = R1 // C
NCH2 = R2 // C
FSPAN = 3200
NFW = R1 // FSPAN
L = 16

f32 = jnp.float32
i32 = jnp.int32


def _med_body(t_ref, o_ref):
    t = t_ref[...]
    arg = t * 1e-5
    T = 10.0 + 5.0 * jnp.sin(arg)
    den = 1.0e4 * (1.0 + 0.1 * jnp.cos(arg))
    logT = jnp.log(T / 300.0)
    nit = -1.0 / T
    z = jnp.zeros_like(T)
    o_ref[...] = jnp.concatenate([logT, nit, den] + [z] * 13, axis=0)


def _flags_body(ph_h, smt_h, phf_h, smtf_h, idx_t, buf_t):
    cid = lax.axis_index("c")
    sid = lax.axis_index("s")
    tid = cid * NS + sid
    iot = lax.iota(i32, L)
    ones = jnp.ones((L,), f32)
    zf = jnp.zeros((L,), f32)

    @pl.when(tid < NFW)
    def _():
        base = tid * FSPAN

        def build(src_h, dst_h):
            pltpu.sync_copy(src_h, idx_t.at[pl.ds(0, NPH)])
            tail0 = (NPH // L) * L
            tv = idx_t[pl.ds(tail0, L)]
            tv = jnp.where(iot < (NPH - tail0), tv, jnp.full((L,), -1, i32))
            idx_t[pl.ds(tail0, L)] = tv

            def zb(i):
                buf_t[pl.ds(pl.multiple_of(i * L, L), L)] = zf
            plsc.parallel_loop(0, FSPAN // L)(zb)

            def bd(i):
                idx = idx_t[pl.ds(pl.multiple_of(i * L, L), L)]
                loc = idx - base
                m = (loc >= 0) & (loc < FSPAN)
                locc = jnp.clip(loc, 0, FSPAN - 1)
                plsc.store_scatter(buf_t, [locc], ones, mask=m)
            plsc.parallel_loop(0, (NPH + L - 1) // L)(bd)
            pltpu.sync_copy(buf_t, dst_h.at[pl.ds(pl.multiple_of(base, 8),
                                                  FSPAN)])

        build(ph_h, phf_h)
        build(smt_h, smtf_h)


_flags_call = pl.kernel(
    _flags_body,
    out_type=(jax.ShapeDtypeStruct((R1,), f32),
              jax.ShapeDtypeStruct((R1,), f32)),
    mesh=plsc.VectorSubcoreMesh(core_axis_name="c", subcore_axis_name="s"),
    compiler_params=pltpu.CompilerParams(needs_layout_passes=False),
    scratch_types=[
        pltpu.VMEM((1024,), i32),
        pltpu.VMEM((FSPAN,), f32),
    ],
)


def _safe_adds(groups):
    for idx, ref, pairs in groups:
        for ofs, val in pairs:
            plsc.addupdate_scatter(ref, [idx + ofs], val)


def _sc_body(y_h, scal_h, alpha_h, beta_h, gamma_h, reac1_h, prod1_h,
             r2a_h, r2b_h, prod2_h, surf_h, mant_h, phf_h, smtf_h,
             out_h,
             y_t, out_t, cnt_t, idx_a, idx_b, idx_c,
             par_a, par_b, par_c,
             flg_a, flg_b, scal_t, smt_t, sem0, sem1):
    cid = lax.axis_index("c")
    sid = lax.axis_index("s")
    tid = cid * NS + sid
    ones = jnp.ones((L,), f32)
    zf = jnp.zeros((L,), f32)
    sems = (sem0, sem1)

    pltpu.sync_copy(y_h.at[pl.ds(tid * (BT * N), BT * N)], y_t)
    pltpu.sync_copy(scal_h.at[pl.ds(tid * (BT * L), BT * L)], scal_t)
    pltpu.sync_copy(surf_h, idx_b.at[pl.ds(0, NSURF)])
    pltpu.sync_copy(mant_h, idx_c.at[pl.ds(0, NMANT)])

    def _zero_range(ref, nsteps):
        def bd(i):
            ref[pl.ds(pl.multiple_of(i * L, L), L)] = zf
        plsc.parallel_loop(0, nsteps)(bd)

    _zero_range(out_t, BT * N // L)
    _zero_range(cnt_t, N // L)

    neg = jnp.full((L,), -1, i32)

    def _sent(i):
        smt_t[pl.ds(pl.multiple_of(i * L, L), L)] = neg
    plsc.parallel_loop(0, 1024 // L)(_sent)

    bofs = [jnp.full((L,), b * N, i32) for b in range(BT)]

    def _ssum(i, acc):
        idx = idx_b[pl.ds(pl.multiple_of(i * L, L), L)]
        plsc.addupdate_scatter(cnt_t, [idx], ones)
        return tuple(acc[b] + plsc.load_gather(y_t, [idx + bofs[b]])
                     for b in range(BT))
    acc_s = plsc.parallel_loop(0, NSURF // L, carry=(zf,) * BT)(_ssum)

    def _msum(i, acc):
        idx = idx_c[pl.ds(pl.multiple_of(i * L, L), L)]
        return tuple(acc[b] + plsc.load_gather(y_t, [idx + bofs[b]])
                     for b in range(BT))
    acc_m = plsc.parallel_loop(0, NMANT // L, carry=(zf,) * BT)(_msum)

    ys = [jnp.full((L,), jnp.sum(acc_s[b]), f32) for b in range(BT)]
    ym = [jnp.full((L,), jnp.sum(acc_m[b]), f32) for b in range(BT)]
    dec = [jnp.minimum(1.0 / (AG * (ys[b] + ym[b])), 1.0) for b in range(BT)]
    srow = [scal_t[pl.ds(b * L, L)] for b in range(BT)]
    Lb = [jnp.full((L,), srow[b][0], f32) for b in range(BT)]
    nit = [jnp.full((L,), srow[b][1], f32) for b in range(BT)]
    den = [jnp.full((L,), srow[b][2], f32) for b in range(BT)]

    def _issue1(g, s):
        off = pl.ds(pl.multiple_of(g * C, 8), C)
        dst = pl.ds(s * C, C)
        sem = sems[s]
        pltpu.async_copy(reac1_h.at[off], idx_a.at[dst], sem)
        pltpu.async_copy(prod1_h.at[off], idx_b.at[dst], sem)
        pltpu.async_copy(alpha_h.at[off], par_a.at[dst], sem)
        pltpu.async_copy(beta_h.at[off], par_b.at[dst], sem)
        pltpu.async_copy(gamma_h.at[off], par_c.at[dst], sem)
        pltpu.async_copy(phf_h.at[off], flg_a.at[dst], sem)
        pltpu.async_copy(smtf_h.at[off], flg_b.at[dst], sem)

    def _drain1(g, s):
        off = pl.ds(pl.multiple_of(g * C, 8), C)
        dst = pl.ds(s * C, C)
        sem = sems[s]
        pltpu.make_async_copy(reac1_h.at[off], idx_a.at[dst], sem).wait()
        pltpu.make_async_copy(prod1_h.at[off], idx_b.at[dst], sem).wait()
        pltpu.make_async_copy(alpha_h.at[off], par_a.at[dst], sem).wait()
        pltpu.make_async_copy(beta_h.at[off], par_b.at[dst], sem).wait()
        pltpu.make_async_copy(gamma_h.at[off], par_c.at[dst], sem).wait()
        pltpu.make_async_copy(phf_h.at[off], flg_a.at[dst], sem).wait()
        pltpu.make_async_copy(smtf_h.at[off], flg_b.at[dst], sem).wait()

    def _issue2(g, s):
        off = pl.ds(pl.multiple_of(g * C, 8), C)
        off2 = pl.ds(pl.multiple_of(R1 + g * C, 8), C)
        dst = pl.ds(s * C, C)
        sem = sems[s]
        pltpu.async_copy(r2a_h.at[off], idx_a.at[dst], sem)
        pltpu.async_copy(r2b_h.at[off], idx_b.at[dst], sem)
        pltpu.async_copy(prod2_h.at[off], idx_c.at[dst], sem)
        pltpu.async_copy(alpha_h.at[off2], par_a.at[dst], sem)
        pltpu.async_copy(beta_h.at[off2], par_b.at[dst], sem)
        pltpu.async_copy(gamma_h.at[off2], par_c.at[dst], sem)

    def _drain2(g, s):
        off = pl.ds(pl.multiple_of(g * C, 8), C)
        off2 = pl.ds(pl.multiple_of(R1 + g * C, 8), C)
        dst = pl.ds(s * C, C)
        sem = sems[s]
        pltpu.make_async_copy(r2a_h.at[off], idx_a.at[dst], sem).wait()
        pltpu.make_async_copy(r2b_h.at[off], idx_b.at[dst], sem).wait()
        pltpu.make_async_copy(prod2_h.at[off], idx_c.at[dst], sem).wait()
        pltpu.make_async_copy(alpha_h.at[off2], par_a.at[dst], sem).wait()
        pltpu.make_async_copy(beta_h.at[off2], par_b.at[dst], sem).wait()
        pltpu.make_async_copy(gamma_h.at[off2], par_c.at[dst], sem).wait()

    def _chunkA1(s, acc):
        def bd(i, acc):
            accg, accl, cap = acc
            off = pl.ds(pl.multiple_of(s * C + i * L, L), L)
            reac = idx_a[off]
            prod = idx_b[off]
            al = par_a[off]
            bt = par_b[off]
            gm = par_c[off]
            phm = flg_a[off] > 0.0
            smm = flg_b[off] > 0.0
            keep = jnp.logical_not(smm)
            wg = plsc.load_gather(cnt_t, [prod])
            wl = plsc.load_gather(cnt_t, [reac])
            ng, nl = [], []
            vp, vr = [], []
            for b in range(BT):
                k = al * jnp.exp(bt * Lb[b] + gm * nit[b])
                k = k * jnp.where(phm, dec[b], ones)
                yv = plsc.load_gather(y_t, [reac + bofs[b]])
                r = k * yv
                ng.append(accg[b] + wg * r)
                nl.append(accl[b] + wl * r)
                vp.append((bofs[b], r))
                vr.append((bofs[b], -r))
            for bo, val in vp:
                plsc.addupdate_scatter(out_t, [prod + bo], val, mask=keep)
            for bo, val in vr:
                plsc.addupdate_scatter(out_t, [reac + bo], val, mask=keep)
            smi = smm.astype(i32)
            cs = plsc.cumsum(smi)
            pos = jnp.clip(cap + cs - 1, 0, 1023)
            packed = reac * 16384 + prod
            plsc.store_scatter(smt_t, [pos], packed, mask=smm)
            pc = plsc.all_reduce_population_count(smm)
            return (tuple(ng), tuple(nl), cap + pc[0])
        return plsc.parallel_loop(0, C // L, carry=acc)(bd)

    acc = ((zf,) * BT, (zf,) * BT, jnp.int32(0))
    _issue1(0, 0)
    _issue1(1, 1)

    def _outerA1(h, acc):
        for s in range(2):
            g = h * 2 + s
            _drain1(g, s)
            acc = _chunkA1(s, acc)

            @pl.when(g + 2 < NCH1)
            def _():
                _issue1(g + 2, s)
        return acc
    acc = lax.fori_loop(0, NCH1 // 2, _outerA1, acc)
    accg1, accl1, _cap = acc
    acc = (accg1, accl1)

    def _chunkA2(s, acc):
        def bd(i, acc):
            accg, accl = acc
            off = pl.ds(pl.multiple_of(s * C + i * L, L), L)
            ra = idx_a[off]
            rb = idx_b[off]
            prd = idx_c[off]
            al = par_a[off]
            bt = par_b[off]
            gm = par_c[off]
            wg = plsc.load_gather(cnt_t, [prd])
            wl = (plsc.load_gather(cnt_t, [ra])
                  + plsc.load_gather(cnt_t, [rb]))
            ng, nl = [], []
            vp, va, vb = [], [], []
            for b in range(BT):
                k = al * jnp.exp(bt * Lb[b] + gm * nit[b]) * den[b]
                ya = plsc.load_gather(y_t, [ra + bofs[b]])
                yb = plsc.load_gather(y_t, [rb + bofs[b]])
                r = k * ya * yb
                ng.append(accg[b] + wg * r)
                nl.append(accl[b] + wl * r)
                rn = -r
                vp.append((bofs[b], r))
                va.append((bofs[b], rn))
                vb.append((bofs[b], rn))
            _safe_adds([(prd, out_t, vp), (ra, out_t, va),
                        (rb, out_t, vb)])
            return (tuple(ng), tuple(nl))
        return plsc.parallel_loop(0, C // L, carry=acc)(bd)

    _issue2(0, 0)
    _issue2(1, 1)

    def _outerA2(h, acc):
        for s in range(2):
            g = h * 2 + s
            _drain2(g, s)
            acc = _chunkA2(s, acc)

            @pl.when(g + 2 < NCH2)
            def _():
                _issue2(g + 2, s)
        return acc
    acc = lax.fori_loop(0, NCH2 // 2, _outerA2, acc)

    accg, accl = acc
    ks = []
    for b in range(BT):
        gv = jnp.full((L,), jnp.sum(accg[b]), f32)
        lv = jnp.full((L,), jnp.sum(accl[b]), f32)
        ks.append(jnp.maximum(gv, 0.0) / (ym[b] + 1e-30)
                  + jnp.maximum(lv, 0.0) / (ys[b] + 1e-30))

    def _fix(i):
        v = smt_t[pl.ds(pl.multiple_of(i * L, L), L)]
        m = v >= 0
        vv = jnp.where(m, v, 0)
        reac = jnp.right_shift(vv, 14)
        prod = jnp.bitwise_and(vv, 16383)
        for b in range(BT):
            yv = plsc.load_gather(y_t, [reac + bofs[b]])
            r = ks[b] * yv
            plsc.addupdate_scatter(out_t, [prod + bofs[b]], r, mask=m)
            plsc.addupdate_scatter(out_t, [reac + bofs[b]], -r, mask=m)
    plsc.parallel_loop(0, 1024 // L)(_fix)

    pltpu.sync_copy(out_t, out_h.at[pl.ds(tid * (BT * N), BT * N)])


_sc_call = pl.kernel(
    _sc_body,
    out_type=jax.ShapeDtypeStruct((B * N,), f32),
    mesh=plsc.VectorSubcoreMesh(core_axis_name="c", subcore_axis_name="s"),
    compiler_params=pltpu.CompilerParams(needs_layout_passes=False),
    scratch_types=[
        pltpu.VMEM((BT * N,), f32),
        pltpu.VMEM((BT * N,), f32),
        pltpu.VMEM((N,), f32),
        pltpu.VMEM((2 * C,), i32),
        pltpu.VMEM((2 * C,), i32),
        pltpu.VMEM((2 * C,), i32),
        pltpu.VMEM((2 * C,), f32),
        pltpu.VMEM((2 * C,), f32),
        pltpu.VMEM((2 * C,), f32),
        pltpu.VMEM((2 * C,), f32),
        pltpu.VMEM((2 * C,), f32),
        pltpu.VMEM((BT * L,), f32),
        pltpu.VMEM((1024,), i32),
        pltpu.SemaphoreType.DMA,
        pltpu.SemaphoreType.DMA,
    ],
)


def kernel(t_in, y_in, alpha, beta, gamma, reac_1st, prod_1st, reac_2nd_a,
           reac_2nd_b, prod_2nd, inds_surf, inds_mant, inds_id_photodeso,
           inds_id_smt):
    scal16 = pl.pallas_call(
        _med_body,
        out_shape=jax.ShapeDtypeStruct((16, B), f32),
    )(t_in.astype(f32).reshape(1, B))
    scal = scal16.T.reshape(-1)

    ph_flags, smt_flags = _flags_call(
        inds_id_photodeso.astype(i32), inds_id_smt.astype(i32))

    out = _sc_call(
        y_in.astype(f32).reshape(-1), scal,
        alpha.astype(f32), beta.astype(f32), gamma.astype(f32),
        reac_1st.astype(i32), prod_1st.astype(i32),
        reac_2nd_a.astype(i32), reac_2nd_b.astype(i32),
        prod_2nd.astype(i32),
        inds_surf.astype(i32), inds_mant.astype(i32),
        ph_flags, smt_flags,
    )
    return out.reshape(B, N)
